# Initial kernel scaffold; baseline (speedup 1.0000x reference)
import jax
import jax.numpy as jnp
from jax.experimental import pallas as pl


def kernel(x, edge_index, batch_ids, conv_w, gru_w_ih, gru_w_hh, gru_b_ih, gru_b_hh, lin0_w, lin0_b, lin1_w, lin1_b):
    raise NotImplementedError("write your pallas kernel here")


# speedup vs baseline: 1.3122x; 1.3122x over previous
import functools

import jax
import jax.numpy as jnp
from jax import lax
from jax.experimental import pallas as pl
from jax.experimental.pallas import tpu as pltpu
from jax.experimental.pallas import tpu_sc as plsc

_N = 10000
_E = 160000
_H = 256
_HH = 128
_NS = 16
_EPT = _E // _NS
_CH = 80
_NCHUNK = _EPT // _CH
_NP = 5000
_DUMP = _NP
_ACC = _NP + _NS
_RPT = 312
_ZR = 104
_TAIL = _NP - _NS * _RPT

_GRID = 10
_R = _N // _GRID


def _make_sc_scatter():
    mesh = plsc.VectorSubcoreMesh(core_axis_name="c", subcore_axis_name="s",
                                  num_cores=1)
    out_type = jax.ShapeDtypeStruct((_N, _HH), jnp.float32)
    scratch = [
        pltpu.VMEM((_NCHUNK, _CH), jnp.int32),
        pltpu.VMEM((_NCHUNK, _CH), jnp.int32),
        pltpu.VMEM((_NCHUNK, _CH), jnp.int32),
        pltpu.VMEM((_NCHUNK, _CH), jnp.int32),
        pltpu.VMEM((_CH, _HH), jnp.float32),
        pltpu.VMEM((_ZR, _HH), jnp.float32),
        pltpu.VMEM_SHARED((_ACC, _HH), jnp.float32),
        pltpu.SemaphoreType.DMA,
    ]

    @functools.partial(pl.kernel, out_type=out_type, mesh=mesh,
                       scratch_types=scratch)
    def sc_scatter(m, src3, dst3, a, src_v, dst_v, dst_lo, dst_hi, rows_v,
                   zbuf, agg_sh, sem):
        s = lax.axis_index("s")

        def zrow(r, carry):
            for j in range(_HH // 16):
                zbuf[r, pl.ds(j * 16, 16)] = jnp.zeros((16,), jnp.float32)
            return carry
        lax.fori_loop(0, _ZR, zrow, 0)

        pltpu.sync_copy(src3.at[s], src_v)
        pltpu.sync_copy(dst3.at[s], dst_v)

        dump = jnp.full((16,), _DUMP, jnp.int32) + s

        def crow(k, carry):
            for j in range(_CH // 16):
                v = dst_v[k, pl.ds(j * 16, 16)]
                dst_lo[k, pl.ds(j * 16, 16)] = jnp.where(v < _NP, v, dump)
                vh = v - _NP
                dst_hi[k, pl.ds(j * 16, 16)] = jnp.where(vh >= 0, vh, dump)
            return carry
        lax.fori_loop(0, _NCHUNK, crow, 0)

        for half, dbuf in ((0, dst_lo), (1, dst_hi)):
            for b in range(_RPT // _ZR):
                pltpu.sync_copy(zbuf,
                                agg_sh.at[pl.ds(s * _RPT + b * _ZR, _ZR)])

            @pl.when(s == _NS - 1)
            def _():
                pltpu.sync_copy(zbuf.at[pl.ds(0, _TAIL)],
                                agg_sh.at[pl.ds(_NS * _RPT, _TAIL)])

            plsc.subcore_barrier()

            def step(k, carry):
                pltpu.async_copy(m.at[src_v.at[k]], rows_v, sem).wait()
                pltpu.sync_copy(rows_v, agg_sh.at[dbuf.at[k]], add=True)
                return carry
            lax.fori_loop(0, _NCHUNK, step, 0)

            plsc.subcore_barrier()

            pltpu.sync_copy(agg_sh.at[pl.ds(s * _RPT, _RPT)],
                            a.at[pl.ds(half * _NP + s * _RPT, _RPT)])

            @pl.when(s == _NS - 1)
            def _():
                pltpu.sync_copy(
                    agg_sh.at[pl.ds(_NS * _RPT, _TAIL)],
                    a.at[pl.ds(half * _NP + _NS * _RPT, _TAIL)])

            plsc.subcore_barrier()

    return sc_scatter


_sc_scatter = _make_sc_scatter()


def _mm_kernel(x_ref, w_ref, m0_ref, m1_ref):
    m = jnp.dot(x_ref[...], w_ref[...], preferred_element_type=jnp.float32)
    m0_ref[...] = m[:, :_HH]
    m1_ref[...] = m[:, _HH:]


def _initial_mm(x, w):
    return pl.pallas_call(
        _mm_kernel,
        grid=(_GRID,),
        in_specs=[
            pl.BlockSpec((_R, _H), lambda i: (i, 0)),
            pl.BlockSpec((_H, _H), lambda i: (0, 0)),
        ],
        out_specs=[pl.BlockSpec((_R, _HH), lambda i: (i, 0))] * 2,
        out_shape=[jax.ShapeDtypeStruct((_N, _HH), jnp.float32)] * 2,
    )(x, w)


def _gru_body(a0_ref, a1_ref, h_ref, wih_ref, whh_ref, bih_ref, bhh_ref,
              *rest, need_m, relu):
    if need_m:
        cw_ref, h_out, m0_ref, m1_ref = rest
    else:
        (h_out,) = rest
    h = h_ref[...]
    agg = jnp.concatenate([a0_ref[...], a1_ref[...]], axis=1)
    gi = jnp.dot(agg, wih_ref[...],
                 preferred_element_type=jnp.float32) + bih_ref[...]
    gh = jnp.dot(h, whh_ref[...],
                 preferred_element_type=jnp.float32) + bhh_ref[...]
    r = jax.nn.sigmoid(gi[:, :_H] + gh[:, :_H])
    z = jax.nn.sigmoid(gi[:, _H:2 * _H] + gh[:, _H:2 * _H])
    n = jnp.tanh(gi[:, 2 * _H:] + r * gh[:, 2 * _H:])
    hn = (1.0 - z) * n + z * h
    if relu:
        hn = jnp.maximum(hn, 0.0)
    h_out[...] = hn
    if need_m:
        m = jnp.dot(hn, cw_ref[...], preferred_element_type=jnp.float32)
        m0_ref[...] = m[:, :_HH]
        m1_ref[...] = m[:, _HH:]


def _gru_step(a0, a1, h, wihT, whhT, bih, bhh, conv_w_next, relu):
    need_m = conv_w_next is not None
    in_specs = [
        pl.BlockSpec((_R, _HH), lambda i: (i, 0)),
        pl.BlockSpec((_R, _HH), lambda i: (i, 0)),
        pl.BlockSpec((_R, _H), lambda i: (i, 0)),
        pl.BlockSpec((_H, 3 * _H), lambda i: (0, 0)),
        pl.BlockSpec((_H, 3 * _H), lambda i: (0, 0)),
        pl.BlockSpec((1, 3 * _H), lambda i: (0, 0)),
        pl.BlockSpec((1, 3 * _H), lambda i: (0, 0)),
    ]
    out_specs = [pl.BlockSpec((_R, _H), lambda i: (i, 0))]
    out_shape = [jax.ShapeDtypeStruct((_N, _H), jnp.float32)]
    args = [a0, a1, h, wihT, whhT, bih, bhh]
    if need_m:
        in_specs.append(pl.BlockSpec((_H, _H), lambda i: (0, 0)))
        args.append(conv_w_next)
        out_specs += [pl.BlockSpec((_R, _HH), lambda i: (i, 0))] * 2
        out_shape += [jax.ShapeDtypeStruct((_N, _HH), jnp.float32)] * 2
    res = pl.pallas_call(
        functools.partial(_gru_body, need_m=need_m, relu=relu),
        grid=(_GRID,),
        in_specs=in_specs,
        out_specs=out_specs,
        out_shape=out_shape,
    )(*args)
    if need_m:
        return res[0], res[1], res[2]
    return res[0], None, None


def _final_body(h_ref, w0_ref, b0_ref, w1_ref, b1_ref, o_ref):
    t = jnp.dot(h_ref[...], w0_ref[...], preferred_element_type=jnp.float32)
    t = jnp.maximum(t + b0_ref[...], 0.0)
    o_ref[...] = jnp.dot(t, w1_ref[...],
                         preferred_element_type=jnp.float32) + b1_ref[...]


def _final(h, lin0T, lin0_b, lin1T, lin1_b):
    return pl.pallas_call(
        _final_body,
        grid=(_GRID,),
        in_specs=[
            pl.BlockSpec((_R, _H), lambda i: (i, 0)),
            pl.BlockSpec((_H, _H), lambda i: (0, 0)),
            pl.BlockSpec((1, _H), lambda i: (0, 0)),
            pl.BlockSpec((_H, _H), lambda i: (0, 0)),
            pl.BlockSpec((1, _H), lambda i: (0, 0)),
        ],
        out_specs=pl.BlockSpec((_R, _H), lambda i: (i, 0)),
        out_shape=jax.ShapeDtypeStruct((_N, _H), jnp.float32),
    )(h, lin0T, lin0_b, lin1T, lin1_b)


_NUM_CONV = 2
_NUM_GRU = 4


def kernel(x, edge_index, batch_ids, conv_w, gru_w_ih, gru_w_hh, gru_b_ih,
           gru_b_hh, lin0_w, lin0_b, lin1_w, lin1_b):
    del batch_ids
    src3 = edge_index[0].reshape(_NS, _NCHUNK, _CH)
    dst3 = edge_index[1].reshape(_NS, _NCHUNK, _CH)
    wihT = jnp.swapaxes(gru_w_ih, 1, 2)
    whhT = jnp.swapaxes(gru_w_hh, 1, 2)
    bih = gru_b_ih.reshape(_NUM_CONV, 1, 3 * _H)
    bhh = gru_b_hh.reshape(_NUM_CONV, 1, 3 * _H)

    h = x.astype(jnp.float32)
    m0, m1 = _initial_mm(h, conv_w[0, 0])
    for l in range(_NUM_CONV):
        for i in range(_NUM_GRU):
            a0 = _sc_scatter(m0, src3, dst3)
            a1 = _sc_scatter(m1, src3, dst3)
            last = (l == _NUM_CONV - 1 and i == _NUM_GRU - 1)
            relu = (l != _NUM_CONV - 1 and i == _NUM_GRU - 1)
            if last:
                w_next = None
            elif i == _NUM_GRU - 1:
                w_next = conv_w[l + 1, 0]
            else:
                w_next = conv_w[l, i + 1]
            h, m0, m1 = _gru_step(a0, a1, h, wihT[l], whhT[l], bih[l], bhh[l],
                                  w_next, relu)
    return _final(h, lin0_w.T, lin0_b.reshape(1, _H), lin1_w.T,
                  lin1_b.reshape(1, _H))


# speedup vs baseline: 3.3256x; 2.5343x over previous
import functools

import jax
import jax.numpy as jnp
from jax import lax
from jax.experimental import pallas as pl
from jax.experimental.pallas import tpu as pltpu
from jax.experimental.pallas import tpu_sc as plsc

_N = 10000
_E = 160000
_H = 256
_HH = 128
_NS = 16
_EPT = 10240
_CH = 128
_NCHUNK = _EPT // _CH
_GRP = 16
_NGRP = _NCHUNK // _GRP
_ACC = _N + _NS
_RPT = 624
_ZR = 48

_GRID = 10
_R = _N // _GRID


def _make_sc_scatter():
    mesh = plsc.VectorSubcoreMesh(core_axis_name="c", subcore_axis_name="s")
    out_type = (jax.ShapeDtypeStruct((_N, _HH), jnp.float32),
                jax.ShapeDtypeStruct((_N, _HH), jnp.float32))
    scratch = [
        pltpu.VMEM((_GRP, _CH), jnp.int32),
        pltpu.VMEM((_GRP, _CH), jnp.int32),
        pltpu.VMEM((_CH, _HH), jnp.float32),
        pltpu.VMEM((_CH, _HH), jnp.float32),
        pltpu.VMEM((_ZR, _HH), jnp.float32),
        pltpu.VMEM_SHARED((_ACC, _HH), jnp.float32),
        pltpu.SemaphoreType.DMA,
        pltpu.SemaphoreType.DMA,
    ]

    @functools.partial(pl.kernel, out_type=out_type, mesh=mesh,
                       scratch_types=scratch)
    def sc_scatter(m0, m1, src3, dst3, a0, a1, src_w, dst_w, rows0, rows1,
                   zbuf, agg_sh, sem0, sem1):
        c = lax.axis_index("c")
        s = lax.axis_index("s")

        def zrow(r, carry):
            for j in range(_HH // 16):
                zbuf[r, pl.ds(j * 16, 16)] = jnp.zeros((16,), jnp.float32)
            return carry
        lax.fori_loop(0, _ZR, zrow, 0)

        for b in range(_RPT // _ZR):
            pltpu.sync_copy(zbuf, agg_sh.at[pl.ds(s * _RPT + b * _ZR, _ZR)])

        @pl.when(s == _NS - 1)
        def _():
            pltpu.sync_copy(zbuf.at[pl.ds(0, _ACC - _NS * _RPT)],
                            agg_sh.at[pl.ds(_NS * _RPT, _ACC - _NS * _RPT)])

        plsc.subcore_barrier()

        rows = (rows0, rows1)
        sems = (sem0, sem1)

        def run(m, a):
            def group(g, carry):
                pltpu.sync_copy(src3.at[s, pl.ds(g * _GRP, _GRP)], src_w)
                pltpu.sync_copy(dst3.at[s, pl.ds(g * _GRP, _GRP)], dst_w)
                pltpu.async_copy(m.at[src_w.at[0]], rows[0], sems[0])
                for r in range(1, _GRP + 1):
                    if r <= _GRP - 1:
                        pltpu.async_copy(m.at[src_w.at[r]], rows[r % 2],
                                         sems[r % 2])
                    b = (r - 1) % 2
                    pltpu.make_async_copy(m.at[src_w.at[r - 1]], rows[b],
                                          sems[b]).wait()
                    pltpu.sync_copy(rows[b], agg_sh.at[dst_w.at[r - 1]],
                                    add=True)
                return carry
            lax.fori_loop(0, _NGRP, group, 0)

        @pl.when(c == 0)
        def _():
            run(m0, a0)

        @pl.when(c == 1)
        def _():
            run(m1, a1)

        plsc.subcore_barrier()

        def wb(a):
            pltpu.sync_copy(agg_sh.at[pl.ds(s * _RPT, _RPT)],
                            a.at[pl.ds(s * _RPT, _RPT)])

            @pl.when(s == _NS - 1)
            def _():
                tail = _N - _NS * _RPT
                pltpu.sync_copy(agg_sh.at[pl.ds(_NS * _RPT, tail)],
                                a.at[pl.ds(_NS * _RPT, tail)])

        @pl.when(c == 0)
        def _():
            wb(a0)

        @pl.when(c == 1)
        def _():
            wb(a1)

    return sc_scatter


_sc_scatter = _make_sc_scatter()


def _mm_kernel(x_ref, w_ref, m0_ref, m1_ref):
    m = jnp.dot(x_ref[...], w_ref[...], preferred_element_type=jnp.float32)
    m0_ref[...] = m[:, :_HH]
    m1_ref[...] = m[:, _HH:]


def _initial_mm(x, w):
    return pl.pallas_call(
        _mm_kernel,
        grid=(_GRID,),
        in_specs=[
            pl.BlockSpec((_R, _H), lambda i: (i, 0)),
            pl.BlockSpec((_H, _H), lambda i: (0, 0)),
        ],
        out_specs=[pl.BlockSpec((_R, _HH), lambda i: (i, 0))] * 2,
        out_shape=[jax.ShapeDtypeStruct((_N, _HH), jnp.float32)] * 2,
    )(x, w)


def _gru_body(a0_ref, a1_ref, h_ref, wih_ref, whh_ref, bih_ref, bhh_ref,
              *rest, need_m, relu):
    if need_m:
        cw_ref, h_out, m0_ref, m1_ref = rest
    else:
        (h_out,) = rest
    h = h_ref[...]
    agg = jnp.concatenate([a0_ref[...], a1_ref[...]], axis=1)
    gi = jnp.dot(agg, wih_ref[...],
                 preferred_element_type=jnp.float32) + bih_ref[...]
    gh = jnp.dot(h, whh_ref[...],
                 preferred_element_type=jnp.float32) + bhh_ref[...]
    r = jax.nn.sigmoid(gi[:, :_H] + gh[:, :_H])
    z = jax.nn.sigmoid(gi[:, _H:2 * _H] + gh[:, _H:2 * _H])
    n = jnp.tanh(gi[:, 2 * _H:] + r * gh[:, 2 * _H:])
    hn = (1.0 - z) * n + z * h
    if relu:
        hn = jnp.maximum(hn, 0.0)
    h_out[...] = hn
    if need_m:
        m = jnp.dot(hn, cw_ref[...], preferred_element_type=jnp.float32)
        m0_ref[...] = m[:, :_HH]
        m1_ref[...] = m[:, _HH:]


def _gru_step(a0, a1, h, wihT, whhT, bih, bhh, conv_w_next, relu):
    need_m = conv_w_next is not None
    in_specs = [
        pl.BlockSpec((_R, _HH), lambda i: (i, 0)),
        pl.BlockSpec((_R, _HH), lambda i: (i, 0)),
        pl.BlockSpec((_R, _H), lambda i: (i, 0)),
        pl.BlockSpec((_H, 3 * _H), lambda i: (0, 0)),
        pl.BlockSpec((_H, 3 * _H), lambda i: (0, 0)),
        pl.BlockSpec((1, 3 * _H), lambda i: (0, 0)),
        pl.BlockSpec((1, 3 * _H), lambda i: (0, 0)),
    ]
    out_specs = [pl.BlockSpec((_R, _H), lambda i: (i, 0))]
    out_shape = [jax.ShapeDtypeStruct((_N, _H), jnp.float32)]
    args = [a0, a1, h, wihT, whhT, bih, bhh]
    if need_m:
        in_specs.append(pl.BlockSpec((_H, _H), lambda i: (0, 0)))
        args.append(conv_w_next)
        out_specs += [pl.BlockSpec((_R, _HH), lambda i: (i, 0))] * 2
        out_shape += [jax.ShapeDtypeStruct((_N, _HH), jnp.float32)] * 2
    res = pl.pallas_call(
        functools.partial(_gru_body, need_m=need_m, relu=relu),
        grid=(_GRID,),
        in_specs=in_specs,
        out_specs=out_specs,
        out_shape=out_shape,
    )(*args)
    if need_m:
        return res[0], res[1], res[2]
    return res[0], None, None


def _final_body(h_ref, w0_ref, b0_ref, w1_ref, b1_ref, o_ref):
    t = jnp.dot(h_ref[...], w0_ref[...], preferred_element_type=jnp.float32)
    t = jnp.maximum(t + b0_ref[...], 0.0)
    o_ref[...] = jnp.dot(t, w1_ref[...],
                         preferred_element_type=jnp.float32) + b1_ref[...]


def _final(h, lin0T, lin0_b, lin1T, lin1_b):
    return pl.pallas_call(
        _final_body,
        grid=(_GRID,),
        in_specs=[
            pl.BlockSpec((_R, _H), lambda i: (i, 0)),
            pl.BlockSpec((_H, _H), lambda i: (0, 0)),
            pl.BlockSpec((1, _H), lambda i: (0, 0)),
            pl.BlockSpec((_H, _H), lambda i: (0, 0)),
            pl.BlockSpec((1, _H), lambda i: (0, 0)),
        ],
        out_specs=pl.BlockSpec((_R, _H), lambda i: (i, 0)),
        out_shape=jax.ShapeDtypeStruct((_N, _H), jnp.float32),
    )(h, lin0T, lin0_b, lin1T, lin1_b)


_NUM_CONV = 2
_NUM_GRU = 4


def kernel(x, edge_index, batch_ids, conv_w, gru_w_ih, gru_w_hh, gru_b_ih,
           gru_b_hh, lin0_w, lin0_b, lin1_w, lin1_b):
    del batch_ids
    pad = _NS * _EPT - _E
    src3 = jnp.concatenate(
        [edge_index[0], jnp.zeros((pad,), jnp.int32)]).reshape(
            _NS, _NCHUNK, _CH)
    dst3 = jnp.concatenate(
        [edge_index[1], jnp.full((pad,), _N, jnp.int32)]).reshape(
            _NS, _NCHUNK, _CH)
    wihT = jnp.swapaxes(gru_w_ih, 1, 2)
    whhT = jnp.swapaxes(gru_w_hh, 1, 2)
    bih = gru_b_ih.reshape(_NUM_CONV, 1, 3 * _H)
    bhh = gru_b_hh.reshape(_NUM_CONV, 1, 3 * _H)

    h = x.astype(jnp.float32)
    m0, m1 = _initial_mm(h, conv_w[0, 0])
    for l in range(_NUM_CONV):
        for i in range(_NUM_GRU):
            a0, a1 = _sc_scatter(m0, m1, src3, dst3)
            last = (l == _NUM_CONV - 1 and i == _NUM_GRU - 1)
            relu = (l != _NUM_CONV - 1 and i == _NUM_GRU - 1)
            if last:
                w_next = None
            elif i == _NUM_GRU - 1:
                w_next = conv_w[l + 1, 0]
            else:
                w_next = conv_w[l, i + 1]
            h, m0, m1 = _gru_step(a0, a1, h, wihT[l], whhT[l], bih[l], bhh[l],
                                  w_next, relu)
    return _final(h, lin0_w.T, lin0_b.reshape(1, _H), lin1_w.T,
                  lin1_b.reshape(1, _H))
